# baseline (speedup 1.0000x reference)
import functools

import jax
import jax.numpy as jnp
import numpy as np
from jax import lax
from jax.experimental import pallas as pl
from jax.experimental.pallas import tpu as pltpu
from jax.experimental.pallas import tpu_sc as plsc

_NC, _NS, _L = 2, 16, 16
_NW = _NC * _NS
_CH = 128
_MAXD = 10


def _p1_body(npad, nch, src3d, bkt_out, be_out, acc_sh, ones_v, idx_all,
             deg_v, bev_all, sem):
    c = lax.axis_index("c")
    s = lax.axis_index("s")
    npt = npad // _NS

    @pl.when(c == 0)
    def _():
        @pl.loop(0, npad // _L)
        def _(i):
            deg_v[pl.ds(i * _L, _L)] = jnp.zeros((_L,), jnp.float32)

        @pl.loop(0, _CH // _L)
        def _(g):
            ones_v[pl.ds(g * _L, _L)] = jnp.ones((_L,), jnp.float32)

        pltpu.sync_copy(deg_v.at[pl.ds(s * npt, npt)],
                        acc_sh.at[pl.ds(s * npt, npt)])
        pltpu.sync_copy(src3d.at[s], idx_all)
        plsc.subcore_barrier()

        @pl.loop(0, nch)
        def _(j):
            pltpu.async_copy(ones_v, acc_sh.at[idx_all.at[j]], sem, add=True)

        @pl.loop(0, nch)
        def _(j):
            pltpu.make_async_copy(ones_v, acc_sh.at[pl.ds(0, _CH)],
                                  sem).wait()

        plsc.subcore_barrier()

        pltpu.sync_copy(acc_sh, deg_v)

        @pl.loop(0, npad // _L)
        def _(i):
            deg_v[pl.ds(i * _L, _L)] = jnp.minimum(
                deg_v[pl.ds(i * _L, _L)], jnp.float32(_MAXD))

        pltpu.sync_copy(deg_v.at[pl.ds(s * npt, npt)],
                        bkt_out.at[pl.ds(s * npt, npt)])

        @pl.loop(0, nch)
        def _(j):
            @pl.loop(0, _CH // _L)
            def _(g):
                sv = idx_all[j, pl.ds(g * _L, _L)]
                bev_all[j, pl.ds(g * _L, _L)] = plsc.load_gather(deg_v, [sv])

        pltpu.sync_copy(bev_all, be_out.at[s])


def _p1_call(src3d, npad):
    nch = src3d.shape[1]
    mesh = plsc.VectorSubcoreMesh(core_axis_name="c", subcore_axis_name="s")
    return pl.kernel(
        functools.partial(_p1_body, npad, nch),
        out_type=(jax.ShapeDtypeStruct((npad,), jnp.float32),
                  jax.ShapeDtypeStruct((_NS, nch, _CH), jnp.float32)),
        mesh=mesh,
        compiler_params=pltpu.CompilerParams(needs_layout_passes=False),
        scratch_types=[
            pltpu.VMEM_SHARED((npad,), jnp.float32),
            pltpu.VMEM((_CH,), jnp.float32),
            pltpu.VMEM((nch, _CH), jnp.int32),
            pltpu.VMEM((npad,), jnp.float32),
            pltpu.VMEM((nch, _CH), jnp.float32),
            pltpu.SemaphoreType.DMA,
        ],
    )(src3d)


_MCH = 64
_MK = 2


def _m_body(t, npad, nch, z_hbm, a_hbm, src3d, dst3d, out_hbm, hacc_sh,
            zb, ab, si, di, di_st, sg, sa, ss):
    c = lax.axis_index("c")
    s = lax.axis_index("s")
    wid = c * _NS + s
    rpt = npad // _NS
    cpr = _CH // _MCH

    @pl.loop(0, _MCH)
    def _(i):
        for seg in range(128 // _L):
            zb[0, i, pl.ds(seg * _L, _L)] = jnp.zeros((_L,), jnp.float32)

    @pl.loop(0, rpt // _MCH)
    def _(k):
        pltpu.sync_copy(zb.at[0], hacc_sh.at[pl.ds(s * rpt + k * _MCH, _MCH)])

    plsc.subcore_barrier()

    pltpu.sync_copy(src3d.at[wid], si)
    pltpu.sync_copy(dst3d.at[wid], di)

    def start_fetch(b, j):
        jrow = j // cpr
        joff = (j % cpr) * _MCH
        pltpu.async_copy(z_hbm.at[si.at[jrow, pl.ds(joff, _MCH)]],
                         zb.at[b], sg[b])
        pltpu.async_copy(a_hbm.at[t, pl.ds((wid * nch + j) * _MCH, _MCH)],
                         ab.at[b], sa[b])
        for g in range(_MCH // _L):
            di_st[b, pl.ds(g * _L, _L)] = di[jrow, pl.ds(joff + g * _L, _L)]

    def wait_fetch(b, j):
        pltpu.make_async_copy(z_hbm.at[si.at[0, pl.ds(0, _MCH)]], zb.at[b],
                              sg[b]).wait()
        pltpu.make_async_copy(a_hbm.at[t, pl.ds(0, _MCH)], ab.at[b],
                              sa[b]).wait()

    def compute(b):
        @pl.loop(0, _MCH)
        def _(i):
            for seg in range(128 // _L):
                zz = zb[b, i, pl.ds(seg * _L, _L)]
                aa = ab[b, i, pl.ds(seg * _L, _L)]
                zb[b, i, pl.ds(seg * _L, _L)] = 1.0 / (1.0 + jnp.exp(zz + aa))

    def start_scatter(b, j):
        pltpu.async_copy(zb.at[b], hacc_sh.at[di_st.at[b]], ss[b], add=True)

    def wait_scatter(b):
        pltpu.make_async_copy(zb.at[b], hacc_sh.at[pl.ds(0, _MCH)],
                              ss[b]).wait()

    for b in range(_MK):
        start_fetch(b, b)

    @pl.loop(0, nch // _MK)
    def _(j2):
        for b in range(_MK):
            j = j2 * _MK + b
            wait_fetch(b, j)
            compute(b)
            start_scatter(b, j)
            pb = (b - 1) % _MK
            jr = j - 1 + _MK

            @pl.when(jnp.logical_and(j >= 1, jr <= nch - 1))
            def _():
                wait_scatter(pb)
                start_fetch(pb, jr)

    for b in range(_MK):
        wait_scatter(b)

    plsc.subcore_barrier()
    pltpu.sync_copy(hacc_sh.at[pl.ds(s * rpt, rpt)],
                    out_hbm.at[c, pl.ds(s * rpt, rpt)])


def _m_call(t, npad, z_neg, a_neg, src3d, dst3d):
    nch = src3d.shape[1] * src3d.shape[2] // _MCH
    mesh = plsc.VectorSubcoreMesh(core_axis_name="c", subcore_axis_name="s")
    return pl.kernel(
        functools.partial(_m_body, t, npad, nch),
        out_type=jax.ShapeDtypeStruct((_NC, npad, 128), jnp.float32),
        mesh=mesh,
        compiler_params=pltpu.CompilerParams(needs_layout_passes=False),
        scratch_types=[
            pltpu.VMEM_SHARED((npad, 128), jnp.float32),
            pltpu.VMEM((_MK, _MCH, 128), jnp.float32),
            pltpu.VMEM((_MK, _MCH, 128), jnp.float32),
            pltpu.VMEM((nch * _MCH // _CH, _CH), jnp.int32),
            pltpu.VMEM((nch * _MCH // _CH, _CH), jnp.int32),
            pltpu.VMEM((_MK, _MCH), jnp.int32),
            [pltpu.SemaphoreType.DMA] * _MK,
            [pltpu.SemaphoreType.DMA] * _MK,
            [pltpu.SemaphoreType.DMA] * _MK,
        ],
    )(z_neg, a_neg, src3d, dst3d)


def _bcol_body(b2_ref, out_ref):
    rows, lanes = b2_ref.shape
    che = rows * lanes
    ei = lax.broadcasted_iota(jnp.int32, (che, rows), 1)
    erow = lax.broadcasted_iota(jnp.int32, (che, rows), 0) // lanes
    e1 = jnp.where(ei == erow, 1.0, 0.0)
    x = lax.dot_general(e1, b2_ref[...], (((1,), (0,)), ((), ())),
                        preferred_element_type=jnp.float32)
    lane = lax.broadcasted_iota(jnp.int32, (che, lanes), 1)
    emod = lax.broadcasted_iota(jnp.int32, (che, lanes), 0) % lanes
    sel = jnp.where(lane == emod, x, 0.0)
    out_ref[...] = jnp.sum(sel, axis=1, keepdims=True).astype(jnp.bfloat16)


def _bcol_call(be2d, rows=16):
    n2, lanes = be2d.shape
    e_pad = n2 * lanes
    return pl.pallas_call(
        _bcol_body,
        grid=(n2 // rows,),
        in_specs=[pl.BlockSpec((rows, lanes), lambda j: (j, 0))],
        out_specs=pl.BlockSpec((rows * lanes, 1), lambda j: (j, 0)),
        out_shape=jax.ShapeDtypeStruct((e_pad, 1), jnp.bfloat16),
    )(be2d)


def _a_body(de, ea_ref, be_ref, wa_ref, out_ref):
    che = ea_ref.shape[0]
    kdim = ea_ref.shape[1]
    bi = jnp.broadcast_to(be_ref[...].astype(jnp.int32), (che, kdim))
    lane_d = lax.broadcasted_iota(jnp.int32, (che, kdim), 1) // de + 1
    x = jnp.where(bi == lane_d, ea_ref[...], jnp.bfloat16(0)
                  ).astype(jnp.float32)
    out_ref[0] = lax.dot_general(x, wa_ref[0], (((1,), (0,)), ((), ())),
                                 preferred_element_type=jnp.float32)


def _a_call(ea160, be, wa_neg, de, che=2048):
    e_pad, kdim = ea160.shape
    t_steps = wa_neg.shape[0]
    return pl.pallas_call(
        functools.partial(_a_body, de),
        grid=(e_pad // che, t_steps),
        in_specs=[
            pl.BlockSpec((che, kdim), lambda j, t: (j, 0)),
            pl.BlockSpec((che, 1), lambda j, t: (j, 0)),
            pl.BlockSpec((1, kdim, 128), lambda j, t: (t, 0, 0)),
        ],
        out_specs=pl.BlockSpec((1, che, 128), lambda j, t: (t, j, 0)),
        out_shape=jax.ShapeDtypeStruct((t_steps, e_pad, 128), jnp.float32),
    )(ea160, be, wa_neg)


def _z_body(p_ref, bkt_ref, wx_ref, out_ref):
    h = p_ref[0] + p_ref[1]
    b = bkt_ref[...]
    x = jnp.concatenate(
        [jnp.where(b == jnp.float32(d), h, 0.0) for d in range(1, _MAXD + 1)],
        axis=1)
    out_ref[...] = lax.dot_general(x, wx_ref[...], (((1,), (0,)), ((), ())),
                                   preferred_element_type=jnp.float32)


def _z_call(p, bkt, wx_t, chn=1024):
    npad = p.shape[1]
    return pl.pallas_call(
        _z_body,
        grid=(npad // chn,),
        in_specs=[
            pl.BlockSpec((2, chn, 128), lambda i: (0, i, 0)),
            pl.BlockSpec((chn, 1), lambda i: (i, 0)),
            pl.BlockSpec((_MAXD * 128, 128), lambda i: (0, 0)),
        ],
        out_specs=pl.BlockSpec((chn, 128), lambda i: (i, 0)),
        out_shape=jax.ShapeDtypeStruct((npad, 128), jnp.float32),
    )(p, bkt, wx_t)


def _r_body(nsteps, t_steps, p_ref, batch_ref, wr_ref, f1w_ref, f1b_ref,
            f2w_ref, f2b_ref, f3w_ref, f3b_ref, out_ref, gacc):
    i = pl.program_id(0)

    @pl.when(i == 0)
    def _():
        gacc[...] = jnp.zeros_like(gacc)

    h = p_ref[0] + p_ref[1]
    logits = lax.dot_general(h, wr_ref[...], (((1,), (0,)), ((), ())),
                             preferred_element_type=jnp.float32)
    o = jnp.zeros((h.shape[0], _MAXD), jnp.float32)
    for t in range(t_steps):
        l = logits[:, t * _MAXD:(t + 1) * _MAXD]
        m = jnp.max(l, axis=1, keepdims=True)
        e = jnp.exp(l - m)
        o = o + e / jnp.sum(e, axis=1, keepdims=True)

    bvec = batch_ref[...]
    gid = lax.broadcasted_iota(jnp.int32, (h.shape[0], gacc.shape[0]), 1)
    oh = jnp.where(bvec == gid, 1.0, 0.0)
    gacc[...] += lax.dot_general(oh, o, (((0,), (0,)), ((), ())),
                                 preferred_element_type=jnp.float32)

    @pl.when(i == nsteps - 1)
    def _():
        g = gacc[...]
        a1 = lax.dot_general(g, f1w_ref[...], (((1,), (1,)), ((), ())),
                             preferred_element_type=jnp.float32) + f1b_ref[...]
        a1 = jnp.where(a1 > 0, a1, 0.01 * a1)
        a2 = lax.dot_general(a1, f2w_ref[...], (((1,), (1,)), ((), ())),
                             preferred_element_type=jnp.float32) + f2b_ref[...]
        a2 = jnp.where(a2 > 0, a2, 0.01 * a2)
        a3 = jnp.sum(a2 * f3w_ref[...], axis=1, keepdims=True) + f3b_ref[0, 0]
        out_ref[...] = jnp.where(a3 > 0, a3, 0.01 * a3)


def _r_call(p, batch2d, wr, f1w, f1b, f2w, f2b, f3w, f3b, chr_=1000):
    n = batch2d.shape[0]
    g = 64
    nsteps = n // chr_
    t_steps = wr.shape[1] // _MAXD
    whole = lambda *shape: pl.BlockSpec(shape, lambda i: tuple(0 for _ in shape))
    return pl.pallas_call(
        functools.partial(_r_body, nsteps, t_steps),
        grid=(nsteps,),
        in_specs=[
            pl.BlockSpec((2, chr_, 128), lambda i: (0, i, 0)),
            pl.BlockSpec((chr_, 1), lambda i: (i, 0)),
            whole(128, wr.shape[1]),
            whole(*f1w.shape),
            whole(*f1b.shape),
            whole(*f2w.shape),
            whole(*f2b.shape),
            whole(*f3w.shape),
            whole(*f3b.shape),
        ],
        out_specs=pl.BlockSpec((g, 1), lambda i: (0, 0)),
        out_shape=jax.ShapeDtypeStruct((g, 1), jnp.float32),
        scratch_shapes=[pltpu.VMEM((g, _MAXD), jnp.float32)],
    )(p, batch2d, wr, f1w, f1b, f2w, f2b, f3w, f3b)


def kernel(x, edge_index, edge_attr, batch, W_msg, W_read,
           fc1_w, fc1_b, fc2_w, fc2_b, fc3_w, fc3_b):
    n, d = x.shape
    e0, de = edge_attr.shape
    t_steps = W_msg.shape[0]

    npad = 10240 if n <= 10240 - _L else ((n + _L + 1279) // 1280) * 1280
    n_dummy = npad - n
    e_tot = e0 + n
    nch_pt = -(-e_tot // (_NW * _MCH))
    _mult = 4 * _MK if _MK % 2 else 2 * _MK
    nch_pt = -(-nch_pt // _mult) * _mult
    e_pad = _NW * nch_pt * _MCH
    pad_cnt = e_pad - e_tot

    loops = jnp.arange(n, dtype=jnp.int32)
    padv = n + (jnp.arange(pad_cnt, dtype=jnp.int32) % n_dummy)
    src = jnp.concatenate([edge_index[0].astype(jnp.int32), loops, padv])
    dst = jnp.concatenate([edge_index[1].astype(jnp.int32), loops, padv])
    src3d = src.reshape(_NW, nch_pt * _MCH // _CH, _CH)
    dst3d = dst.reshape(_NW, nch_pt * _MCH // _CH, _CH)
    src3dp = src.reshape(_NS, -1, _CH)
    ea_pad = jnp.concatenate(
        [edge_attr, jnp.zeros((e_pad - e0, de), edge_attr.dtype)])

    bkt1d, be3d = _p1_call(src3dp, npad)
    bkt = bkt1d.reshape(npad, 1)
    be = _bcol_call(be3d.reshape(e_pad // _CH, _CH))

    wa_neg = -W_msg[:, :, d:, :].reshape(t_steps, _MAXD * de, 128)
    ea160 = jnp.tile(ea_pad.astype(jnp.bfloat16), (1, _MAXD))
    a_negs = [_a_call(ea160, be, wa_neg[t:t + 1], de) for t in range(t_steps)]

    wx = -W_msg[:, :, :d, :].reshape(t_steps, _MAXD * d, 128)
    x_pad = jnp.pad(x, ((0, n_dummy), (0, 0)))
    p = jnp.stack([x_pad, jnp.zeros_like(x_pad)])
    for t in range(t_steps):
        z_neg = _z_call(p, bkt, wx[t])
        p = _m_call(0, npad, z_neg, a_negs[t], src3d, dst3d)

    wr = jnp.transpose(W_read, (1, 0, 2)).reshape(d, t_steps * _MAXD)
    out = _r_call(p, batch.reshape(n, 1).astype(jnp.int32), wr,
                  fc1_w, fc1_b.reshape(1, -1), fc2_w, fc2_b.reshape(1, -1),
                  fc3_w, fc3_b.reshape(1, -1))
    return out


# speedup vs baseline: 1.3464x; 1.3464x over previous
import functools

import jax
import jax.numpy as jnp
import numpy as np
from jax import lax
from jax.experimental import pallas as pl
from jax.experimental.pallas import tpu as pltpu
from jax.experimental.pallas import tpu_sc as plsc

_NC, _NS, _L = 2, 16, 16
_NW = _NC * _NS
_CH = 128
_MAXD = 10


def _p1_body(npad, nch, src3d, bkt_out, be_out, acc_sh, ones_v, idx_all,
             deg_v, bev_all, sem):
    c = lax.axis_index("c")
    s = lax.axis_index("s")
    npt = npad // _NS

    @pl.when(c == 0)
    def _():
        @pl.loop(0, npad // _L)
        def _(i):
            deg_v[pl.ds(i * _L, _L)] = jnp.zeros((_L,), jnp.float32)

        @pl.loop(0, _CH // _L)
        def _(g):
            ones_v[pl.ds(g * _L, _L)] = jnp.ones((_L,), jnp.float32)

        pltpu.sync_copy(deg_v.at[pl.ds(s * npt, npt)],
                        acc_sh.at[pl.ds(s * npt, npt)])
        pltpu.sync_copy(src3d.at[s], idx_all)
        plsc.subcore_barrier()

        @pl.loop(0, nch)
        def _(j):
            pltpu.async_copy(ones_v, acc_sh.at[idx_all.at[j]], sem, add=True)

        @pl.loop(0, nch)
        def _(j):
            pltpu.make_async_copy(ones_v, acc_sh.at[pl.ds(0, _CH)],
                                  sem).wait()

        plsc.subcore_barrier()

        pltpu.sync_copy(acc_sh, deg_v)

        @pl.loop(0, npad // _L)
        def _(i):
            deg_v[pl.ds(i * _L, _L)] = jnp.minimum(
                deg_v[pl.ds(i * _L, _L)], jnp.float32(_MAXD))

        pltpu.sync_copy(deg_v.at[pl.ds(s * npt, npt)],
                        bkt_out.at[pl.ds(s * npt, npt)])

        @pl.loop(0, nch)
        def _(j):
            @pl.loop(0, _CH // _L)
            def _(g):
                sv = idx_all[j, pl.ds(g * _L, _L)]
                bev_all[j, pl.ds(g * _L, _L)] = plsc.load_gather(deg_v, [sv])

        pltpu.sync_copy(bev_all, be_out.at[s])


def _p1_call(src3d, npad):
    nch = src3d.shape[1]
    mesh = plsc.VectorSubcoreMesh(core_axis_name="c", subcore_axis_name="s")
    return pl.kernel(
        functools.partial(_p1_body, npad, nch),
        out_type=(jax.ShapeDtypeStruct((npad,), jnp.float32),
                  jax.ShapeDtypeStruct((_NS, nch, _CH), jnp.float32)),
        mesh=mesh,
        compiler_params=pltpu.CompilerParams(needs_layout_passes=False),
        scratch_types=[
            pltpu.VMEM_SHARED((npad,), jnp.float32),
            pltpu.VMEM((_CH,), jnp.float32),
            pltpu.VMEM((nch, _CH), jnp.int32),
            pltpu.VMEM((npad,), jnp.float32),
            pltpu.VMEM((nch, _CH), jnp.float32),
            pltpu.SemaphoreType.DMA,
        ],
    )(src3d)


_MCH = 32
_MK = 4


def _m_body(t, npad, nch, z_hbm, a_hbm, src3d, dst3d, out_hbm, hacc_sh,
            zb, ab, si, di, di_st, sg, sa, ss):
    c = lax.axis_index("c")
    s = lax.axis_index("s")
    wid = c * _NS + s
    rpt = npad // _NS
    cpr = _CH // _MCH

    @pl.loop(0, _MCH)
    def _(i):
        for seg in range(128 // _L):
            zb[0, i, pl.ds(seg * _L, _L)] = jnp.zeros((_L,), jnp.float32)

    @pl.loop(0, rpt // _MCH)
    def _(k):
        pltpu.sync_copy(zb.at[0], hacc_sh.at[pl.ds(s * rpt + k * _MCH, _MCH)])

    plsc.subcore_barrier()

    pltpu.sync_copy(src3d.at[wid], si)
    pltpu.sync_copy(dst3d.at[wid], di)

    def start_fetch(b, j):
        jrow = j // cpr
        joff = (j % cpr) * _MCH
        pltpu.async_copy(z_hbm.at[si.at[jrow, pl.ds(joff, _MCH)]],
                         zb.at[b], sg[b])
        pltpu.async_copy(a_hbm.at[t, pl.ds((wid * nch + j) * _MCH, _MCH)],
                         ab.at[b], sa[b])
        for g in range(_MCH // _L):
            di_st[b, pl.ds(g * _L, _L)] = di[jrow, pl.ds(joff + g * _L, _L)]

    def wait_fetch(b, j):
        pltpu.make_async_copy(z_hbm.at[si.at[0, pl.ds(0, _MCH)]], zb.at[b],
                              sg[b]).wait()
        pltpu.make_async_copy(a_hbm.at[t, pl.ds(0, _MCH)], ab.at[b],
                              sa[b]).wait()

    def compute(b):
        @pl.loop(0, _MCH)
        def _(i):
            for seg in range(128 // _L):
                zz = zb[b, i, pl.ds(seg * _L, _L)]
                aa = ab[b, i, pl.ds(seg * _L, _L)]
                zb[b, i, pl.ds(seg * _L, _L)] = 1.0 / (1.0 + jnp.exp(zz + aa))

    def start_scatter(b, j):
        pltpu.async_copy(zb.at[b], hacc_sh.at[di_st.at[b]], ss[b], add=True)

    def wait_scatter(b):
        pltpu.make_async_copy(zb.at[b], hacc_sh.at[pl.ds(0, _MCH)],
                              ss[b]).wait()

    for b in range(_MK):
        start_fetch(b, b)

    @pl.loop(0, nch // _MK)
    def _(j2):
        for b in range(_MK):
            j = j2 * _MK + b
            wait_fetch(b, j)
            compute(b)
            start_scatter(b, j)
            pb = (b - 1) % _MK
            jr = j - 1 + _MK

            @pl.when(jnp.logical_and(j >= 1, jr <= nch - 1))
            def _():
                wait_scatter(pb)
                start_fetch(pb, jr)

    for b in range(_MK):
        wait_scatter(b)

    plsc.subcore_barrier()
    pltpu.sync_copy(hacc_sh.at[pl.ds(s * rpt, rpt)],
                    out_hbm.at[c, pl.ds(s * rpt, rpt)])


def _m_call(t, npad, z_neg, a_neg, src3d, dst3d):
    nch = src3d.shape[1] * src3d.shape[2] // _MCH
    mesh = plsc.VectorSubcoreMesh(core_axis_name="c", subcore_axis_name="s")
    return pl.kernel(
        functools.partial(_m_body, t, npad, nch),
        out_type=jax.ShapeDtypeStruct((_NC, npad, 128), jnp.float32),
        mesh=mesh,
        compiler_params=pltpu.CompilerParams(needs_layout_passes=False),
        scratch_types=[
            pltpu.VMEM_SHARED((npad, 128), jnp.float32),
            pltpu.VMEM((_MK, _MCH, 128), jnp.float32),
            pltpu.VMEM((_MK, _MCH, 128), jnp.float32),
            pltpu.VMEM((nch * _MCH // _CH, _CH), jnp.int32),
            pltpu.VMEM((nch * _MCH // _CH, _CH), jnp.int32),
            pltpu.VMEM((_MK, _MCH), jnp.int32),
            [pltpu.SemaphoreType.DMA] * _MK,
            [pltpu.SemaphoreType.DMA] * _MK,
            [pltpu.SemaphoreType.DMA] * _MK,
        ],
    )(z_neg, a_neg, src3d, dst3d)


def _bcol_body(b2_ref, out_ref):
    rows, lanes = b2_ref.shape
    che = rows * lanes
    ei = lax.broadcasted_iota(jnp.int32, (che, rows), 1)
    erow = lax.broadcasted_iota(jnp.int32, (che, rows), 0) // lanes
    e1 = jnp.where(ei == erow, 1.0, 0.0)
    x = lax.dot_general(e1, b2_ref[...], (((1,), (0,)), ((), ())),
                        preferred_element_type=jnp.float32)
    lane = lax.broadcasted_iota(jnp.int32, (che, lanes), 1)
    emod = lax.broadcasted_iota(jnp.int32, (che, lanes), 0) % lanes
    sel = jnp.where(lane == emod, x, 0.0)
    out_ref[...] = jnp.sum(sel, axis=1, keepdims=True).astype(jnp.bfloat16)


def _bcol_call(be2d, rows=16):
    n2, lanes = be2d.shape
    e_pad = n2 * lanes
    return pl.pallas_call(
        _bcol_body,
        grid=(n2 // rows,),
        in_specs=[pl.BlockSpec((rows, lanes), lambda j: (j, 0))],
        out_specs=pl.BlockSpec((rows * lanes, 1), lambda j: (j, 0)),
        out_shape=jax.ShapeDtypeStruct((e_pad, 1), jnp.bfloat16),
    )(be2d)


def _a_body(de, ea_ref, be_ref, wa_ref, out_ref):
    che = ea_ref.shape[0]
    kdim = ea_ref.shape[1]
    bi = jnp.broadcast_to(be_ref[...].astype(jnp.int32), (che, kdim))
    lane_d = lax.broadcasted_iota(jnp.int32, (che, kdim), 1) // de + 1
    x = jnp.where(bi == lane_d, ea_ref[...], jnp.bfloat16(0)
                  ).astype(jnp.float32)
    out_ref[0] = lax.dot_general(x, wa_ref[0], (((1,), (0,)), ((), ())),
                                 preferred_element_type=jnp.float32)


def _a_call(ea160, be, wa_neg, de, che=2048):
    e_pad, kdim = ea160.shape
    t_steps = wa_neg.shape[0]
    return pl.pallas_call(
        functools.partial(_a_body, de),
        grid=(e_pad // che, t_steps),
        in_specs=[
            pl.BlockSpec((che, kdim), lambda j, t: (j, 0)),
            pl.BlockSpec((che, 1), lambda j, t: (j, 0)),
            pl.BlockSpec((1, kdim, 128), lambda j, t: (t, 0, 0)),
        ],
        out_specs=pl.BlockSpec((1, che, 128), lambda j, t: (t, j, 0)),
        out_shape=jax.ShapeDtypeStruct((t_steps, e_pad, 128), jnp.float32),
    )(ea160, be, wa_neg)


def _z_body(p_ref, bkt_ref, wx_ref, out_ref):
    h = p_ref[0] + p_ref[1]
    b = bkt_ref[...]
    x = jnp.concatenate(
        [jnp.where(b == jnp.float32(d), h, 0.0) for d in range(1, _MAXD + 1)],
        axis=1)
    out_ref[...] = lax.dot_general(x, wx_ref[...], (((1,), (0,)), ((), ())),
                                   preferred_element_type=jnp.float32)


def _z_call(p, bkt, wx_t, chn=1024):
    npad = p.shape[1]
    return pl.pallas_call(
        _z_body,
        grid=(npad // chn,),
        in_specs=[
            pl.BlockSpec((2, chn, 128), lambda i: (0, i, 0)),
            pl.BlockSpec((chn, 1), lambda i: (i, 0)),
            pl.BlockSpec((_MAXD * 128, 128), lambda i: (0, 0)),
        ],
        out_specs=pl.BlockSpec((chn, 128), lambda i: (i, 0)),
        out_shape=jax.ShapeDtypeStruct((npad, 128), jnp.float32),
    )(p, bkt, wx_t)


def _r_body(nsteps, t_steps, p_ref, batch_ref, wr_ref, f1w_ref, f1b_ref,
            f2w_ref, f2b_ref, f3w_ref, f3b_ref, out_ref, gacc):
    i = pl.program_id(0)

    @pl.when(i == 0)
    def _():
        gacc[...] = jnp.zeros_like(gacc)

    h = p_ref[0] + p_ref[1]
    logits = lax.dot_general(h, wr_ref[...], (((1,), (0,)), ((), ())),
                             preferred_element_type=jnp.float32)
    o = jnp.zeros((h.shape[0], _MAXD), jnp.float32)
    for t in range(t_steps):
        l = logits[:, t * _MAXD:(t + 1) * _MAXD]
        m = jnp.max(l, axis=1, keepdims=True)
        e = jnp.exp(l - m)
        o = o + e / jnp.sum(e, axis=1, keepdims=True)

    bvec = batch_ref[...]
    gid = lax.broadcasted_iota(jnp.int32, (h.shape[0], gacc.shape[0]), 1)
    oh = jnp.where(bvec == gid, 1.0, 0.0)
    gacc[...] += lax.dot_general(oh, o, (((0,), (0,)), ((), ())),
                                 preferred_element_type=jnp.float32)

    @pl.when(i == nsteps - 1)
    def _():
        g = gacc[...]
        a1 = lax.dot_general(g, f1w_ref[...], (((1,), (1,)), ((), ())),
                             preferred_element_type=jnp.float32) + f1b_ref[...]
        a1 = jnp.where(a1 > 0, a1, 0.01 * a1)
        a2 = lax.dot_general(a1, f2w_ref[...], (((1,), (1,)), ((), ())),
                             preferred_element_type=jnp.float32) + f2b_ref[...]
        a2 = jnp.where(a2 > 0, a2, 0.01 * a2)
        a3 = jnp.sum(a2 * f3w_ref[...], axis=1, keepdims=True) + f3b_ref[0, 0]
        out_ref[...] = jnp.where(a3 > 0, a3, 0.01 * a3)


def _r_call(p, batch2d, wr, f1w, f1b, f2w, f2b, f3w, f3b, chr_=1000):
    n = batch2d.shape[0]
    g = 64
    nsteps = n // chr_
    t_steps = wr.shape[1] // _MAXD
    whole = lambda *shape: pl.BlockSpec(shape, lambda i: tuple(0 for _ in shape))
    return pl.pallas_call(
        functools.partial(_r_body, nsteps, t_steps),
        grid=(nsteps,),
        in_specs=[
            pl.BlockSpec((2, chr_, 128), lambda i: (0, i, 0)),
            pl.BlockSpec((chr_, 1), lambda i: (i, 0)),
            whole(128, wr.shape[1]),
            whole(*f1w.shape),
            whole(*f1b.shape),
            whole(*f2w.shape),
            whole(*f2b.shape),
            whole(*f3w.shape),
            whole(*f3b.shape),
        ],
        out_specs=pl.BlockSpec((g, 1), lambda i: (0, 0)),
        out_shape=jax.ShapeDtypeStruct((g, 1), jnp.float32),
        scratch_shapes=[pltpu.VMEM((g, _MAXD), jnp.float32)],
    )(p, batch2d, wr, f1w, f1b, f2w, f2b, f3w, f3b)


def kernel(x, edge_index, edge_attr, batch, W_msg, W_read,
           fc1_w, fc1_b, fc2_w, fc2_b, fc3_w, fc3_b):
    n, d = x.shape
    e0, de = edge_attr.shape
    t_steps = W_msg.shape[0]

    npad = 10240 if n <= 10240 - _L else ((n + _L + 1279) // 1280) * 1280
    n_dummy = npad - n
    e_tot = e0 + n
    nch_pt = -(-e_tot // (_NW * _MCH))
    _mult = 4 * _MK if _MK % 2 else 2 * _MK
    nch_pt = -(-nch_pt // _mult) * _mult
    e_pad = _NW * nch_pt * _MCH
    pad_cnt = e_pad - e_tot

    loops = jnp.arange(n, dtype=jnp.int32)
    padv = n + (jnp.arange(pad_cnt, dtype=jnp.int32) % n_dummy)
    src = jnp.concatenate([edge_index[0].astype(jnp.int32), loops, padv])
    dst = jnp.concatenate([edge_index[1].astype(jnp.int32), loops, padv])
    src3d = src.reshape(_NW, nch_pt * _MCH // _CH, _CH)
    dst3d = dst.reshape(_NW, nch_pt * _MCH // _CH, _CH)
    src3dp = src.reshape(_NS, -1, _CH)
    ea_pad = jnp.concatenate(
        [edge_attr, jnp.zeros((e_pad - e0, de), edge_attr.dtype)])

    bkt1d, be3d = _p1_call(src3dp, npad)
    bkt = bkt1d.reshape(npad, 1)
    be = _bcol_call(be3d.reshape(e_pad // _CH, _CH))

    wa_neg = -W_msg[:, :, d:, :].reshape(t_steps, _MAXD * de, 128)
    ea160 = jnp.tile(ea_pad.astype(jnp.bfloat16), (1, _MAXD))
    a_negs = [_a_call(ea160, be, wa_neg[t:t + 1], de) for t in range(t_steps)]

    wx = -W_msg[:, :, :d, :].reshape(t_steps, _MAXD * d, 128)
    x_pad = jnp.pad(x, ((0, n_dummy), (0, 0)))
    p = jnp.stack([x_pad, jnp.zeros_like(x_pad)])
    for t in range(t_steps):
        z_neg = _z_call(p, bkt, wx[t])
        p = _m_call(0, npad, z_neg, a_negs[t], src3d, dst3d)

    wr = jnp.transpose(W_read, (1, 0, 2)).reshape(d, t_steps * _MAXD)
    out = _r_call(p, batch.reshape(n, 1).astype(jnp.int32), wr,
                  fc1_w, fc1_b.reshape(1, -1), fc2_w, fc2_b.reshape(1, -1),
                  fc3_w, fc3_b.reshape(1, -1))
    return out
